# baseline (speedup 1.0000x reference)
import functools

import numpy as np
import jax
import jax.numpy as jnp
from jax import lax
from jax.experimental import pallas as pl
from jax.experimental.pallas import tpu as pltpu
from jax.experimental.pallas import tpu_sc as plsc

_D = 128
_SLICE = 128 * 128
_NVOX = _D * _SLICE
_NW = 32
_L = 16
_SPW = _D // _NW
_NIDS = 48
_TAB = 64

_F0, _F1, _F2 = True, False, True
_SUBN_DATA = np.array([3647288517, 4265293960], np.uint32)
_KEY0_DATA = np.array([1889313301, 2441599006], np.uint32)
_PERM = np.array([11, 38, 5, 16, 46, 45, 47, 7, 39, 15, 1, 2, 40, 8, 43, 27,
                  24, 32, 23, 36, 10, 28, 37, 42, 35, 14, 17, 13, 0, 9, 6, 12,
                  25, 41, 34, 19, 3, 20, 44, 4, 31, 22, 33, 30, 29, 26, 21,
                  18], np.int32)


def _presence_body(labels_hbm, out_hbm, labs_v, pres_v):
    c = lax.axis_index("c")
    s = lax.axis_index("s")
    wid = s * 2 + c
    zeros = jnp.zeros((_L,), jnp.float32)
    for i in range(_TAB // _L):
        pres_v[pl.ds(i * _L, _L)] = zeros
    ones = jnp.ones((_L,), jnp.float32)

    for k in range(_SPW):
        d = wid * _SPW + k
        pltpu.sync_copy(labels_hbm.at[d], labs_v)

        def row_body(g, carry):
            base = g * 128
            for j in range(8):
                labs = labs_v[pl.ds(base + j * _L, _L)]
                plsc.store_scatter(pres_v, [labs], ones)
            return carry

        lax.fori_loop(0, _SLICE // 128, row_body, 0)

    pltpu.sync_copy(pres_v, out_hbm.at[wid])


def _tables(present):
    n = jnp.sum(present)
    sub_n = jax.random.wrap_key_data(jnp.asarray(_SUBN_DATA))
    n_hide = jax.random.randint(sub_n, (), n // 2, n - 1)
    hidden = jnp.zeros(_NIDS, bool).at[jnp.asarray(_PERM)].set(
        jnp.arange(_NIDS) < n_hide)
    kept = present & (~hidden)
    key0 = jax.random.wrap_key_data(jnp.asarray(_KEY0_DATA))

    def step(key, keptu):
        def _draw(k):
            k, k1, k2 = jax.random.split(k, 3)
            b = jax.random.bernoulli(k1, 0.5)
            lo = jnp.where(b, 0.0, 1.9)
            hi = jnp.where(b, 0.1, 2.0)
            val = jax.random.uniform(k2, (), minval=lo, maxval=hi)
            return k, b, val

        def _skip(k):
            return k, jnp.zeros((), bool), jnp.zeros((), jnp.result_type(float))

        key, b, val = lax.cond(keptu, _draw, _skip, key)
        return key, (b, val)

    _, (bs, vals) = lax.scan(step, key0, kept[1:_NIDS])
    b_full = jnp.concatenate([jnp.zeros(1, bool), bs])
    v_full = jnp.concatenate([jnp.zeros(1, vals.dtype), vals])
    u = jnp.arange(_NIDS)
    m = kept & (u >= 1)
    scal_tab = jnp.where(m, v_full, 1.0).astype(jnp.float32)
    code_tab = jnp.where(m, jnp.where(b_full, 1.0, 2.0), 0.0).astype(
        jnp.float32)
    tabs = jnp.zeros(2 * _TAB, jnp.float32)
    tabs = tabs.at[0:_NIDS].set(scal_tab)
    tabs = tabs.at[_TAB:_TAB + _NIDS].set(code_tab)
    return tabs


def _fill_body(lab_hbm, tabs_hbm, scal_hbm, oh_hbm,
               labs_v, tabs_v, s_v, o1_v, o2_v, bg_v):
    c = lax.axis_index("c")
    s = lax.axis_index("s")
    wid = s * 2 + c
    pltpu.sync_copy(tabs_hbm, tabs_v)
    iota = lax.iota(jnp.int32, _L)
    one = jnp.float32(1.0)
    zero = jnp.float32(0.0)

    def gather_chunk(src_base, j):
        if _F2:
            idx = (src_base + 127 - j * _L) - iota
        else:
            idx = (src_base + j * _L) + iota
        labs = plsc.load_gather(labs_v, [idx])
        return labs

    di0 = (_D - 1) if _F0 else 0
    pltpu.sync_copy(lab_hbm.at[di0], labs_v)

    def bg_row(r, carry):
        ir = (127 - r) if _F1 else r
        src_base = ir * 128
        q = r * 128
        for j in range(8):
            labs = gather_chunk(src_base, j)
            code = plsc.load_gather(tabs_v, [labs + _TAB])
            bg_v[pl.ds(q + j * _L, _L)] = jnp.where(code == zero, one, zero)
        return carry

    lax.fori_loop(0, 128, bg_row, 0)

    for k in range(_SPW):
        d = wid * _SPW + k
        di = (127 - d) if _F0 else d
        pltpu.sync_copy(lab_hbm.at[di], labs_v)

        def row_body(r, carry):
            ir = (127 - r) if _F1 else r
            src_base = ir * 128
            q = r * 128
            for j in range(8):
                labs = gather_chunk(src_base, j)
                sv = plsc.load_gather(tabs_v, [labs])
                code = plsc.load_gather(tabs_v, [labs + _TAB])
                qq = q + j * _L
                s_v[pl.ds(qq, _L)] = sv
                o1_v[pl.ds(qq, _L)] = jnp.where(code == one, one, zero)
                o2_v[pl.ds(qq, _L)] = jnp.where(code == 2.0, one, zero)
            return carry

        lax.fori_loop(0, 128, row_body, 0)
        pltpu.sync_copy(s_v, scal_hbm.at[d])
        pltpu.sync_copy(o1_v, oh_hbm.at[1, d])
        pltpu.sync_copy(o2_v, oh_hbm.at[2, d])
        pltpu.sync_copy(bg_v, oh_hbm.at[0, d])


@functools.lru_cache(maxsize=None)
def _build_kernels():
    mesh = plsc.VectorSubcoreMesh(
        core_axis_name="c", subcore_axis_name="s",
        num_cores=2, num_subcores=16)
    params = pltpu.CompilerParams(needs_layout_passes=False)
    presence = pl.kernel(
        _presence_body,
        out_type=jax.ShapeDtypeStruct((_NW, _TAB), jnp.float32),
        mesh=mesh,
        scratch_types=[
            pltpu.VMEM((_SLICE,), jnp.int32),
            pltpu.VMEM((_TAB,), jnp.float32),
        ],
        compiler_params=params,
    )
    fill = pl.kernel(
        _fill_body,
        out_type=(
            jax.ShapeDtypeStruct((_D, _SLICE), jnp.float32),
            jax.ShapeDtypeStruct((3, _D, _SLICE), jnp.float32),
        ),
        mesh=mesh,
        scratch_types=[
            pltpu.VMEM((_SLICE,), jnp.int32),
            pltpu.VMEM((2 * _TAB,), jnp.float32),
            pltpu.VMEM((_SLICE,), jnp.float32),
            pltpu.VMEM((_SLICE,), jnp.float32),
            pltpu.VMEM((_SLICE,), jnp.float32),
            pltpu.VMEM((_SLICE,), jnp.float32),
        ],
        compiler_params=params,
    )
    return presence, fill


def kernel(vessel_labels):
    presence_kernel, fill_kernel = _build_kernels()
    lab2 = vessel_labels.reshape(_D, _SLICE)
    pres_part = presence_kernel(lab2)
    present = jnp.any(pres_part[:, :_NIDS] > 0.0, axis=0)
    tabs = _tables(present)
    scal, oh = fill_kernel(lab2, tabs)
    return scal.reshape(_D, 128, 128), oh.reshape(3, _D, 128, 128)


# speedup vs baseline: 1.2302x; 1.2302x over previous
import functools

import numpy as np
import jax
import jax.numpy as jnp
from jax import lax
from jax.experimental import pallas as pl
from jax.experimental.pallas import tpu as pltpu
from jax.experimental.pallas import tpu_sc as plsc

_D = 128
_SLICE = 128 * 128
_NVOX = _D * _SLICE
_NW = 32
_L = 16
_SPW = _D // _NW
_NIDS = 48
_TAB = 64

_F0, _F1, _F2 = True, False, True
_SUBN_DATA = np.array([3647288517, 4265293960], np.uint32)
_KEY0_DATA = np.array([1889313301, 2441599006], np.uint32)
_PERM = np.array([11, 38, 5, 16, 46, 45, 47, 7, 39, 15, 1, 2, 40, 8, 43, 27,
                  24, 32, 23, 36, 10, 28, 37, 42, 35, 14, 17, 13, 0, 9, 6, 12,
                  25, 41, 34, 19, 3, 20, 44, 4, 31, 22, 33, 30, 29, 26, 21,
                  18], np.int32)


def _presence_body(labels_hbm, out_hbm, labs_v, pres_v):
    c = lax.axis_index("c")
    s = lax.axis_index("s")
    wid = s * 2 + c
    zeros = jnp.zeros((_L,), jnp.float32)
    for i in range(_TAB // _L):
        pres_v[pl.ds(i * _L, _L)] = zeros
    ones = jnp.ones((_L,), jnp.float32)

    for k in range(_SPW):
        d = wid * _SPW + k
        pltpu.sync_copy(labels_hbm.at[d], labs_v)

        @plsc.parallel_loop(0, _SLICE // 128, unroll=4)
        def row_body(g):
            base = g * 128
            for j in range(8):
                labs = labs_v[pl.ds(base + j * _L, _L)]
                plsc.store_scatter(pres_v, [labs], ones)

    pltpu.sync_copy(pres_v, out_hbm.at[wid])


def _tables(present):
    n = jnp.sum(present)
    sub_n = jax.random.wrap_key_data(jnp.asarray(_SUBN_DATA))
    n_hide = jax.random.randint(sub_n, (), n // 2, n - 1)
    hidden = jnp.zeros(_NIDS, bool).at[jnp.asarray(_PERM)].set(
        jnp.arange(_NIDS) < n_hide)
    kept = present & (~hidden)
    key0 = jax.random.wrap_key_data(jnp.asarray(_KEY0_DATA))

    def step(key, keptu):
        def _draw(k):
            k, k1, k2 = jax.random.split(k, 3)
            b = jax.random.bernoulli(k1, 0.5)
            lo = jnp.where(b, 0.0, 1.9)
            hi = jnp.where(b, 0.1, 2.0)
            val = jax.random.uniform(k2, (), minval=lo, maxval=hi)
            return k, b, val

        def _skip(k):
            return k, jnp.zeros((), bool), jnp.zeros((), jnp.result_type(float))

        key, b, val = lax.cond(keptu, _draw, _skip, key)
        return key, (b, val)

    _, (bs, vals) = lax.scan(step, key0, kept[1:_NIDS])
    b_full = jnp.concatenate([jnp.zeros(1, bool), bs])
    v_full = jnp.concatenate([jnp.zeros(1, vals.dtype), vals])
    u = jnp.arange(_NIDS)
    m = kept & (u >= 1)
    scal_tab = jnp.where(m, v_full, 1.0).astype(jnp.float32)
    code_tab = jnp.where(m, jnp.where(b_full, 1.0, 2.0), 0.0).astype(
        jnp.float32)
    tabs = jnp.zeros(2 * _TAB, jnp.float32)
    tabs = tabs.at[0:_NIDS].set(scal_tab)
    tabs = tabs.at[_TAB:_TAB + _NIDS].set(code_tab)
    return tabs


def _fill_body(lab_hbm, tabs_hbm, scal_hbm, oh_hbm,
               labs_v, tabs_v, s_v, o1_v, o2_v, bg_v):
    c = lax.axis_index("c")
    s = lax.axis_index("s")
    wid = s * 2 + c
    pltpu.sync_copy(tabs_hbm, tabs_v)
    iota = lax.iota(jnp.int32, _L)
    one = jnp.float32(1.0)
    zero = jnp.float32(0.0)

    def gather_chunk(src_base, j):
        if _F2:
            idx = (src_base + 127 - j * _L) - iota
        else:
            idx = (src_base + j * _L) + iota
        labs = plsc.load_gather(labs_v, [idx])
        return labs

    di0 = (_D - 1) if _F0 else 0
    pltpu.sync_copy(lab_hbm.at[di0], labs_v)

    @plsc.parallel_loop(0, 128, unroll=4)
    def bg_row(r):
        ir = (127 - r) if _F1 else r
        src_base = ir * 128
        q = r * 128
        for j in range(8):
            labs = gather_chunk(src_base, j)
            code = plsc.load_gather(tabs_v, [labs + _TAB])
            bg_v[pl.ds(q + j * _L, _L)] = jnp.where(code == zero, one, zero)

    for k in range(_SPW):
        d = wid * _SPW + k
        di = (127 - d) if _F0 else d
        pltpu.sync_copy(lab_hbm.at[di], labs_v)

        @plsc.parallel_loop(0, 128, unroll=4)
        def row_body(r):
            ir = (127 - r) if _F1 else r
            src_base = ir * 128
            q = r * 128
            for j in range(8):
                labs = gather_chunk(src_base, j)
                sv = plsc.load_gather(tabs_v, [labs])
                code = plsc.load_gather(tabs_v, [labs + _TAB])
                qq = q + j * _L
                s_v[pl.ds(qq, _L)] = sv
                o1_v[pl.ds(qq, _L)] = jnp.where(code == one, one, zero)
                o2_v[pl.ds(qq, _L)] = jnp.where(code == 2.0, one, zero)
        pltpu.sync_copy(s_v, scal_hbm.at[d])
        pltpu.sync_copy(o1_v, oh_hbm.at[1, d])
        pltpu.sync_copy(o2_v, oh_hbm.at[2, d])
        pltpu.sync_copy(bg_v, oh_hbm.at[0, d])


@functools.lru_cache(maxsize=None)
def _build_kernels():
    mesh = plsc.VectorSubcoreMesh(
        core_axis_name="c", subcore_axis_name="s",
        num_cores=2, num_subcores=16)
    params = pltpu.CompilerParams(needs_layout_passes=False)
    presence = pl.kernel(
        _presence_body,
        out_type=jax.ShapeDtypeStruct((_NW, _TAB), jnp.float32),
        mesh=mesh,
        scratch_types=[
            pltpu.VMEM((_SLICE,), jnp.int32),
            pltpu.VMEM((_TAB,), jnp.float32),
        ],
        compiler_params=params,
    )
    fill = pl.kernel(
        _fill_body,
        out_type=(
            jax.ShapeDtypeStruct((_D, _SLICE), jnp.float32),
            jax.ShapeDtypeStruct((3, _D, _SLICE), jnp.float32),
        ),
        mesh=mesh,
        scratch_types=[
            pltpu.VMEM((_SLICE,), jnp.int32),
            pltpu.VMEM((2 * _TAB,), jnp.float32),
            pltpu.VMEM((_SLICE,), jnp.float32),
            pltpu.VMEM((_SLICE,), jnp.float32),
            pltpu.VMEM((_SLICE,), jnp.float32),
            pltpu.VMEM((_SLICE,), jnp.float32),
        ],
        compiler_params=params,
    )
    return presence, fill


def kernel(vessel_labels):
    presence_kernel, fill_kernel = _build_kernels()
    lab2 = vessel_labels.reshape(_D, _SLICE)
    pres_part = presence_kernel(lab2)
    present = jnp.any(pres_part[:, :_NIDS] > 0.0, axis=0)
    tabs = _tables(present)
    scal, oh = fill_kernel(lab2, tabs)
    return scal.reshape(_D, 128, 128), oh.reshape(3, _D, 128, 128)


# speedup vs baseline: 1.4432x; 1.1732x over previous
import functools

import numpy as np
import jax
import jax.numpy as jnp
from jax import lax
from jax.experimental import pallas as pl
from jax.experimental.pallas import tpu as pltpu
from jax.experimental.pallas import tpu_sc as plsc

_D = 128
_R = 128
_C = 128
_NW = 32
_L = 16
_SPW = _D // _NW
_NIDS = 48
_TAB = 64

_F0, _F1, _F2 = True, False, True
_SUBN_DATA = np.array([3647288517, 4265293960], np.uint32)
_KEY0_DATA = np.array([1889313301, 2441599006], np.uint32)
_PERM = np.array([11, 38, 5, 16, 46, 45, 47, 7, 39, 15, 1, 2, 40, 8, 43, 27,
                  24, 32, 23, 36, 10, 28, 37, 42, 35, 14, 17, 13, 0, 9, 6, 12,
                  25, 41, 34, 19, 3, 20, 44, 4, 31, 22, 33, 30, 29, 26, 21,
                  18], np.int32)


def _presence_body(labels_hbm, out_hbm, labs_v, pres_v):
    c = lax.axis_index("c")
    s = lax.axis_index("s")
    wid = s * 2 + c
    zeros = jnp.zeros((_L,), jnp.float32)
    for i in range(_TAB // _L):
        pres_v[pl.ds(i * _L, _L)] = zeros
    ones = jnp.ones((_L,), jnp.float32)

    for k in range(_SPW):
        d = wid * _SPW + k
        pltpu.sync_copy(labels_hbm.at[d], labs_v)

        @plsc.parallel_loop(0, _R, unroll=4)
        def row_body(r):
            for j in range(_C // _L):
                labs = labs_v[r, pl.ds(j * _L, _L)]
                plsc.store_scatter(pres_v, [labs], ones)

    pltpu.sync_copy(pres_v, out_hbm.at[wid])


def _tables(present):
    n = jnp.sum(present)
    sub_n = jax.random.wrap_key_data(jnp.asarray(_SUBN_DATA))
    n_hide = jax.random.randint(sub_n, (), n // 2, n - 1)
    hidden = jnp.zeros(_NIDS, bool).at[jnp.asarray(_PERM)].set(
        jnp.arange(_NIDS) < n_hide)
    kept = present & (~hidden)
    key0 = jax.random.wrap_key_data(jnp.asarray(_KEY0_DATA))

    def step(key, keptu):
        def _draw(k):
            k, k1, k2 = jax.random.split(k, 3)
            b = jax.random.bernoulli(k1, 0.5)
            lo = jnp.where(b, 0.0, 1.9)
            hi = jnp.where(b, 0.1, 2.0)
            val = jax.random.uniform(k2, (), minval=lo, maxval=hi)
            return k, b, val

        def _skip(k):
            return k, jnp.zeros((), bool), jnp.zeros((), jnp.result_type(float))

        key, b, val = lax.cond(keptu, _draw, _skip, key)
        return key, (b, val)

    _, (bs, vals) = lax.scan(step, key0, kept[1:_NIDS])
    b_full = jnp.concatenate([jnp.zeros(1, bool), bs])
    v_full = jnp.concatenate([jnp.zeros(1, vals.dtype), vals])
    u = jnp.arange(_NIDS)
    m = kept & (u >= 1)
    scal_tab = jnp.where(m, v_full, 1.0).astype(jnp.float32)
    code_tab = jnp.where(m, jnp.where(b_full, 1.0, 2.0), 0.0).astype(
        jnp.float32)
    tabs = jnp.zeros(2 * _TAB, jnp.float32)
    tabs = tabs.at[0:_NIDS].set(scal_tab)
    tabs = tabs.at[_TAB:_TAB + _NIDS].set(code_tab)
    return tabs


def _fill_body(lab_hbm, tabs_hbm, scal_hbm, oh_hbm,
               labs_v, tabs_v, s_v, o1_v, o2_v, bg_v):
    c = lax.axis_index("c")
    s = lax.axis_index("s")
    wid = s * 2 + c
    pltpu.sync_copy(tabs_hbm, tabs_v)
    iota = lax.iota(jnp.int32, _L)
    one = jnp.float32(1.0)
    zero = jnp.float32(0.0)

    def gather_labs(ir, j):
        if _F2:
            col = (127 - j * _L) - iota
        else:
            col = (j * _L) + iota
        row = jnp.full((_L,), ir, jnp.int32)
        return plsc.load_gather(labs_v, [row, col])

    di0 = (_D - 1) if _F0 else 0
    pltpu.sync_copy(lab_hbm.at[di0], labs_v)

    @plsc.parallel_loop(0, _R, unroll=4)
    def bg_row(r):
        ir = (127 - r) if _F1 else r
        for j in range(_C // _L):
            labs = gather_labs(ir, j)
            code = plsc.load_gather(tabs_v, [labs + _TAB])
            bg_v[r, pl.ds(j * _L, _L)] = jnp.where(code == zero, one, zero)

    for k in range(_SPW):
        d = wid * _SPW + k
        di = (127 - d) if _F0 else d
        pltpu.sync_copy(lab_hbm.at[di], labs_v)

        @plsc.parallel_loop(0, _R, unroll=4)
        def row_body(r):
            ir = (127 - r) if _F1 else r
            for j in range(_C // _L):
                labs = gather_labs(ir, j)
                sv = plsc.load_gather(tabs_v, [labs])
                code = plsc.load_gather(tabs_v, [labs + _TAB])
                cs = pl.ds(j * _L, _L)
                s_v[r, cs] = sv
                o1_v[r, cs] = jnp.where(code == one, one, zero)
                o2_v[r, cs] = jnp.where(code == 2.0, one, zero)

        pltpu.sync_copy(s_v, scal_hbm.at[d])
        pltpu.sync_copy(o1_v, oh_hbm.at[1, d])
        pltpu.sync_copy(o2_v, oh_hbm.at[2, d])
        pltpu.sync_copy(bg_v, oh_hbm.at[0, d])


@functools.lru_cache(maxsize=None)
def _build_kernels():
    mesh = plsc.VectorSubcoreMesh(
        core_axis_name="c", subcore_axis_name="s",
        num_cores=2, num_subcores=16)
    params = pltpu.CompilerParams(needs_layout_passes=False)
    presence = pl.kernel(
        _presence_body,
        out_type=jax.ShapeDtypeStruct((_NW, _TAB), jnp.float32),
        mesh=mesh,
        scratch_types=[
            pltpu.VMEM((_R, _C), jnp.int32),
            pltpu.VMEM((_TAB,), jnp.float32),
        ],
        compiler_params=params,
    )
    fill = pl.kernel(
        _fill_body,
        out_type=(
            jax.ShapeDtypeStruct((_D, _R, _C), jnp.float32),
            jax.ShapeDtypeStruct((3, _D, _R, _C), jnp.float32),
        ),
        mesh=mesh,
        scratch_types=[
            pltpu.VMEM((_R, _C), jnp.int32),
            pltpu.VMEM((2 * _TAB,), jnp.float32),
            pltpu.VMEM((_R, _C), jnp.float32),
            pltpu.VMEM((_R, _C), jnp.float32),
            pltpu.VMEM((_R, _C), jnp.float32),
            pltpu.VMEM((_R, _C), jnp.float32),
        ],
        compiler_params=params,
    )
    return presence, fill


def kernel(vessel_labels):
    presence_kernel, fill_kernel = _build_kernels()
    pres_part = presence_kernel(vessel_labels)
    present = jnp.any(pres_part[:, :_NIDS] > 0.0, axis=0)
    tabs = _tables(present)
    scal, oh = fill_kernel(vessel_labels, tabs)
    return scal, oh


# speedup vs baseline: 4.1223x; 2.8563x over previous
import functools

import numpy as np
import jax
import jax.numpy as jnp
from jax import lax
from jax.experimental import pallas as pl
from jax.experimental.pallas import tpu as pltpu
from jax.experimental.pallas import tpu_sc as plsc

_D = 128
_R = 128
_C = 128
_NW = 32
_L = 16
_SPW = _D // _NW
_NIDS = 48
_TAB = 64

_F0, _F1, _F2 = True, False, True
_SUBN_DATA = np.array([3647288517, 4265293960], np.uint32)
_KEY0_DATA = np.array([1889313301, 2441599006], np.uint32)
_PERM = np.array([11, 38, 5, 16, 46, 45, 47, 7, 39, 15, 1, 2, 40, 8, 43, 27,
                  24, 32, 23, 36, 10, 28, 37, 42, 35, 14, 17, 13, 0, 9, 6, 12,
                  25, 41, 34, 19, 3, 20, 44, 4, 31, 22, 33, 30, 29, 26, 21,
                  18], np.int32)
_B_ALL = np.array([0, 0, 1, 0, 0, 1, 1, 1, 0, 1, 1, 0, 1, 0, 0, 0, 1, 1, 0,
                   0, 0, 0, 1, 1, 0, 0, 0, 1, 0, 0, 0, 0, 1, 1, 1, 1, 0, 0,
                   0, 0, 0, 0, 1, 0, 1, 1, 1], bool)
_V_ALL = np.array([
    1.9962396621704102, 1.9997227191925049, 0.011770523153245449,
    1.9923416376113892, 1.9883031845092773, 0.025162935256958008,
    0.01231150608509779, 0.05545854568481445, 1.9056047201156616,
    0.005695521831512451, 0.08964892476797104, 1.901131510734558,
    0.09200332313776016, 1.9152088165283203, 1.9811745882034302,
    1.9047857522964478, 0.0292035099118948, 0.08672022074460983,
    1.9730931520462036, 1.9855785369873047, 1.9920490980148315,
    1.9331375360488892, 0.017901409417390823, 0.007629287429153919,
    1.9230473041534424, 1.9163955450057983, 1.9104191064834595,
    0.008841372095048428, 1.9622188806533813, 1.9610812664031982,
    1.940725564956665, 1.9660009145736694, 0.057649850845336914,
    0.03789961338043213, 0.021052313968539238, 0.099941186606884,
    1.9690383672714233, 1.9550373554229736, 1.9016425609588623,
    1.9830209016799927, 1.9710121154785156, 1.9140957593917847,
    0.004095733165740967, 1.981557846069336, 0.08646737784147263,
    0.067454993724823, 0.09673076122999191], np.float32)


def _presence_body(labels_hbm, out_hbm, labs_v, pres_v):
    c = lax.axis_index("c")
    s = lax.axis_index("s")
    wid = s * 2 + c
    zeros = jnp.zeros((_L,), jnp.float32)
    for i in range(_TAB // _L):
        pres_v[pl.ds(i * _L, _L)] = zeros
    ones = jnp.ones((_L,), jnp.float32)

    for k in range(_SPW):
        d = wid * _SPW + k
        pltpu.sync_copy(labels_hbm.at[d], labs_v)

        @plsc.parallel_loop(0, _R, unroll=4)
        def row_body(r):
            for j in range(_C // _L):
                labs = labs_v[r, pl.ds(j * _L, _L)]
                plsc.store_scatter(pres_v, [labs], ones)

    pltpu.sync_copy(pres_v, out_hbm.at[wid])


def _tables(present):
    n = jnp.sum(present)
    sub_n = jax.random.wrap_key_data(jnp.asarray(_SUBN_DATA))
    n_hide = jax.random.randint(sub_n, (), n // 2, n - 1)
    hidden = jnp.zeros(_NIDS, bool).at[jnp.asarray(_PERM)].set(
        jnp.arange(_NIDS) < n_hide)
    kept = present & (~hidden)
    k47 = kept[1:_NIDS]
    ki = k47.astype(jnp.int32)
    rank = jnp.cumsum(ki) - ki
    bs = jnp.asarray(_B_ALL)[rank]
    vals = jnp.asarray(_V_ALL)[rank]
    b_full = jnp.concatenate([jnp.zeros(1, bool), bs])
    v_full = jnp.concatenate([jnp.zeros(1, vals.dtype), vals])
    u = jnp.arange(_NIDS)
    m = kept & (u >= 1)
    scal_tab = jnp.where(m, v_full, 1.0).astype(jnp.float32)
    code_tab = jnp.where(m, jnp.where(b_full, 1.0, 2.0), 0.0).astype(
        jnp.float32)
    tabs = jnp.zeros(2 * _TAB, jnp.float32)
    tabs = tabs.at[0:_NIDS].set(scal_tab)
    tabs = tabs.at[_TAB:_TAB + _NIDS].set(code_tab)
    return tabs


def _fill_body(lab_hbm, tabs_hbm, scal_hbm, oh_hbm,
               labs_v, tabs_v, s_v, o1_v, o2_v, bg_v):
    c = lax.axis_index("c")
    s = lax.axis_index("s")
    wid = s * 2 + c
    pltpu.sync_copy(tabs_hbm, tabs_v)
    iota = lax.iota(jnp.int32, _L)
    one = jnp.float32(1.0)
    zero = jnp.float32(0.0)

    def gather_labs(ir, j):
        if _F2:
            col = (127 - j * _L) - iota
        else:
            col = (j * _L) + iota
        row = jnp.full((_L,), ir, jnp.int32)
        return plsc.load_gather(labs_v, [row, col])

    di0 = (_D - 1) if _F0 else 0
    pltpu.sync_copy(lab_hbm.at[di0], labs_v)

    @plsc.parallel_loop(0, _R, unroll=4)
    def bg_row(r):
        ir = (127 - r) if _F1 else r
        for j in range(_C // _L):
            labs = gather_labs(ir, j)
            code = plsc.load_gather(tabs_v, [labs + _TAB])
            bg_v[r, pl.ds(j * _L, _L)] = jnp.where(code == zero, one, zero)

    for k in range(_SPW):
        d = wid * _SPW + k
        di = (127 - d) if _F0 else d
        pltpu.sync_copy(lab_hbm.at[di], labs_v)

        @plsc.parallel_loop(0, _R, unroll=4)
        def row_body(r):
            ir = (127 - r) if _F1 else r
            for j in range(_C // _L):
                labs = gather_labs(ir, j)
                sv = plsc.load_gather(tabs_v, [labs])
                code = plsc.load_gather(tabs_v, [labs + _TAB])
                cs = pl.ds(j * _L, _L)
                s_v[r, cs] = sv
                o1_v[r, cs] = jnp.where(code == one, one, zero)
                o2_v[r, cs] = jnp.where(code == 2.0, one, zero)

        pltpu.sync_copy(s_v, scal_hbm.at[d])
        pltpu.sync_copy(o1_v, oh_hbm.at[1, d])
        pltpu.sync_copy(o2_v, oh_hbm.at[2, d])
        pltpu.sync_copy(bg_v, oh_hbm.at[0, d])


@functools.lru_cache(maxsize=None)
def _build_kernels():
    mesh = plsc.VectorSubcoreMesh(
        core_axis_name="c", subcore_axis_name="s",
        num_cores=2, num_subcores=16)
    params = pltpu.CompilerParams(needs_layout_passes=False)
    presence = pl.kernel(
        _presence_body,
        out_type=jax.ShapeDtypeStruct((_NW, _TAB), jnp.float32),
        mesh=mesh,
        scratch_types=[
            pltpu.VMEM((_R, _C), jnp.int32),
            pltpu.VMEM((_TAB,), jnp.float32),
        ],
        compiler_params=params,
    )
    fill = pl.kernel(
        _fill_body,
        out_type=(
            jax.ShapeDtypeStruct((_D, _R, _C), jnp.float32),
            jax.ShapeDtypeStruct((3, _D, _R, _C), jnp.float32),
        ),
        mesh=mesh,
        scratch_types=[
            pltpu.VMEM((_R, _C), jnp.int32),
            pltpu.VMEM((2 * _TAB,), jnp.float32),
            pltpu.VMEM((_R, _C), jnp.float32),
            pltpu.VMEM((_R, _C), jnp.float32),
            pltpu.VMEM((_R, _C), jnp.float32),
            pltpu.VMEM((_R, _C), jnp.float32),
        ],
        compiler_params=params,
    )
    return presence, fill


def kernel(vessel_labels):
    presence_kernel, fill_kernel = _build_kernels()
    pres_part = presence_kernel(vessel_labels)
    present = jnp.any(pres_part[:, :_NIDS] > 0.0, axis=0)
    tabs = _tables(present)
    scal, oh = fill_kernel(vessel_labels, tabs)
    return scal, oh


# speedup vs baseline: 4.5753x; 1.1099x over previous
import functools

import numpy as np
import jax
import jax.numpy as jnp
from jax import lax
from jax.experimental import pallas as pl
from jax.experimental.pallas import tpu as pltpu
from jax.experimental.pallas import tpu_sc as plsc

_D = 128
_R = 128
_C = 128
_NW = 32
_L = 16
_SPW = _D // _NW
_NIDS = 48
_TAB = 64

_F0, _F1, _F2 = True, False, True
_SUBN_DATA = np.array([3647288517, 4265293960], np.uint32)
_KEY0_DATA = np.array([1889313301, 2441599006], np.uint32)
_PERM = np.array([11, 38, 5, 16, 46, 45, 47, 7, 39, 15, 1, 2, 40, 8, 43, 27,
                  24, 32, 23, 36, 10, 28, 37, 42, 35, 14, 17, 13, 0, 9, 6, 12,
                  25, 41, 34, 19, 3, 20, 44, 4, 31, 22, 33, 30, 29, 26, 21,
                  18], np.int32)
_B_ALL = np.array([0, 0, 1, 0, 0, 1, 1, 1, 0, 1, 1, 0, 1, 0, 0, 0, 1, 1, 0,
                   0, 0, 0, 1, 1, 0, 0, 0, 1, 0, 0, 0, 0, 1, 1, 1, 1, 0, 0,
                   0, 0, 0, 0, 1, 0, 1, 1, 1], bool)
_V_ALL = np.array([
    1.9962396621704102, 1.9997227191925049, 0.011770523153245449,
    1.9923416376113892, 1.9883031845092773, 0.025162935256958008,
    0.01231150608509779, 0.05545854568481445, 1.9056047201156616,
    0.005695521831512451, 0.08964892476797104, 1.901131510734558,
    0.09200332313776016, 1.9152088165283203, 1.9811745882034302,
    1.9047857522964478, 0.0292035099118948, 0.08672022074460983,
    1.9730931520462036, 1.9855785369873047, 1.9920490980148315,
    1.9331375360488892, 0.017901409417390823, 0.007629287429153919,
    1.9230473041534424, 1.9163955450057983, 1.9104191064834595,
    0.008841372095048428, 1.9622188806533813, 1.9610812664031982,
    1.940725564956665, 1.9660009145736694, 0.057649850845336914,
    0.03789961338043213, 0.021052313968539238, 0.099941186606884,
    1.9690383672714233, 1.9550373554229736, 1.9016425609588623,
    1.9830209016799927, 1.9710121154785156, 1.9140957593917847,
    0.004095733165740967, 1.981557846069336, 0.08646737784147263,
    0.067454993724823, 0.09673076122999191], np.float32)


def _presence_body(labels_hbm, out_hbm, labs_v, pres_v):
    c = lax.axis_index("c")
    s = lax.axis_index("s")
    wid = s * 2 + c
    zeros = jnp.zeros((_L,), jnp.float32)
    for i in range(_TAB // _L):
        pres_v[pl.ds(i * _L, _L)] = zeros
    ones = jnp.ones((_L,), jnp.float32)

    for k in range(_SPW):
        d = wid * _SPW + k
        pltpu.sync_copy(labels_hbm.at[d], labs_v)

        @plsc.parallel_loop(0, _R, unroll=4)
        def row_body(r):
            for j in range(_C // _L):
                labs = labs_v[r, pl.ds(j * _L, _L)]
                plsc.store_scatter(pres_v, [labs], ones)

    pltpu.sync_copy(pres_v, out_hbm.at[wid])


def _tables(present):
    n = jnp.sum(present)
    sub_n = jax.random.wrap_key_data(jnp.asarray(_SUBN_DATA))
    n_hide = jax.random.randint(sub_n, (), n // 2, n - 1)
    hidden = jnp.zeros(_NIDS, bool).at[jnp.asarray(_PERM)].set(
        jnp.arange(_NIDS) < n_hide)
    kept = present & (~hidden)
    k47 = kept[1:_NIDS]
    ki = k47.astype(jnp.int32)
    rank = jnp.cumsum(ki) - ki
    bs = jnp.asarray(_B_ALL)[rank]
    vals = jnp.asarray(_V_ALL)[rank]
    b_full = jnp.concatenate([jnp.zeros(1, bool), bs])
    v_full = jnp.concatenate([jnp.zeros(1, vals.dtype), vals])
    u = jnp.arange(_NIDS)
    m = kept & (u >= 1)
    scal_tab = jnp.where(m, v_full, 1.0).astype(jnp.float32)
    code_tab = jnp.where(m, jnp.where(b_full, 1.0, 2.0), 0.0).astype(
        jnp.float32)
    tabs = jnp.zeros(2 * _TAB, jnp.float32)
    tabs = tabs.at[0:_NIDS].set(scal_tab)
    tabs = tabs.at[_TAB:_TAB + _NIDS].set(code_tab)
    return tabs


_HR = _R // 2


def _fill_body(lab_hbm, tabs_hbm, scal_hbm, oh_hbm,
               labs_a, labs_b, tabs_v, s_a, s_b, o1_a, o1_b, o2_a, o2_b,
               bg_v, sem_in_a, sem_in_b, sem_out_a, sem_out_b, sem_bg):
    c = lax.axis_index("c")
    s = lax.axis_index("s")
    wid = s * 2 + c
    pltpu.sync_copy(tabs_hbm, tabs_v)
    iota = lax.iota(jnp.int32, _L)
    one = jnp.float32(1.0)
    zero = jnp.float32(0.0)
    labs = (labs_a, labs_b)
    s_ = (s_a, s_b)
    o1_ = (o1_a, o1_b)
    o2_ = (o2_a, o2_b)
    sem_in = (sem_in_a, sem_in_b)
    sem_out = (sem_out_a, sem_out_b)

    def gather_labs(labs_v, lir, j):
        if _F2:
            col = (127 - j * _L) - iota
        else:
            col = (j * _L) + iota
        row = jnp.full((_L,), lir, jnp.int32)
        return plsc.load_gather(labs_v, [row, col])

    def in_half(h):
        if _F1:
            return 1 - h, lambda lr: (_HR - 1) - lr
        return h, lambda lr: lr

    di0 = (_D - 1) if _F0 else 0
    for h in range(2):
        hh, lmap = in_half(h)
        pltpu.sync_copy(lab_hbm.at[di0, pl.ds(hh * _HR, _HR)], labs_a)

        @plsc.parallel_loop(0, _HR, unroll=4)
        def bg_row(lr):
            lir = lmap(lr)
            for j in range(_C // _L):
                lv = gather_labs(labs_a, lir, j)
                code = plsc.load_gather(tabs_v, [lv + _TAB])
                bg_v[h * _HR + lr, pl.ds(j * _L, _L)] = jnp.where(
                    code == zero, one, zero)

    bg_descs = [
        pltpu.async_copy(bg_v, oh_hbm.at[0, wid * _SPW + k], sem_bg)
        for k in range(_SPW)
    ]

    nu = 2 * _SPW

    def start_load(u):
        k, h = u >> 1, u & 1
        d = wid * _SPW + k
        di = (127 - d) if _F0 else d
        hh, _ = in_half(h)
        return pltpu.async_copy(
            lab_hbm.at[di, pl.ds(hh * _HR, _HR)], labs[u & 1], sem_in[u & 1])

    in_d = {0: start_load(0)}
    out_d = {}
    for u in range(nu):
        buf = u & 1
        if u + 1 < nu:
            in_d[u + 1] = start_load(u + 1)
        in_d[u].wait()
        if u >= 2:
            for dsc in out_d[u - 2]:
                dsc.wait()
        k, h = u >> 1, u & 1
        _, lmap = in_half(h)

        @plsc.parallel_loop(0, _HR, unroll=4)
        def row_body(lr):
            lir = lmap(lr)
            for j in range(_C // _L):
                lv = gather_labs(labs[buf], lir, j)
                sv = plsc.load_gather(tabs_v, [lv])
                code = plsc.load_gather(tabs_v, [lv + _TAB])
                cs = pl.ds(j * _L, _L)
                s_[buf][lr, cs] = sv
                o1_[buf][lr, cs] = jnp.where(code == one, one, zero)
                o2_[buf][lr, cs] = jnp.where(code == 2.0, one, zero)

        d = wid * _SPW + k
        hs = pl.ds(h * _HR, _HR)
        out_d[u] = [
            pltpu.async_copy(s_[buf], scal_hbm.at[d, hs], sem_out[buf]),
            pltpu.async_copy(o1_[buf], oh_hbm.at[1, d, hs], sem_out[buf]),
            pltpu.async_copy(o2_[buf], oh_hbm.at[2, d, hs], sem_out[buf]),
        ]

    for u in (nu - 2, nu - 1):
        for dsc in out_d[u]:
            dsc.wait()
    for dsc in bg_descs:
        dsc.wait()


@functools.lru_cache(maxsize=None)
def _build_kernels():
    mesh = plsc.VectorSubcoreMesh(
        core_axis_name="c", subcore_axis_name="s",
        num_cores=2, num_subcores=16)
    params = pltpu.CompilerParams(needs_layout_passes=False)
    presence = pl.kernel(
        _presence_body,
        out_type=jax.ShapeDtypeStruct((_NW, _TAB), jnp.float32),
        mesh=mesh,
        scratch_types=[
            pltpu.VMEM((_R, _C), jnp.int32),
            pltpu.VMEM((_TAB,), jnp.float32),
        ],
        compiler_params=params,
    )
    fill = pl.kernel(
        _fill_body,
        out_type=(
            jax.ShapeDtypeStruct((_D, _R, _C), jnp.float32),
            jax.ShapeDtypeStruct((3, _D, _R, _C), jnp.float32),
        ),
        mesh=mesh,
        scratch_types=[
            pltpu.VMEM((_HR, _C), jnp.int32),
            pltpu.VMEM((_HR, _C), jnp.int32),
            pltpu.VMEM((2 * _TAB,), jnp.float32),
            pltpu.VMEM((_HR, _C), jnp.float32),
            pltpu.VMEM((_HR, _C), jnp.float32),
            pltpu.VMEM((_HR, _C), jnp.float32),
            pltpu.VMEM((_HR, _C), jnp.float32),
            pltpu.VMEM((_HR, _C), jnp.float32),
            pltpu.VMEM((_HR, _C), jnp.float32),
            pltpu.VMEM((_R, _C), jnp.float32),
            pltpu.SemaphoreType.DMA,
            pltpu.SemaphoreType.DMA,
            pltpu.SemaphoreType.DMA,
            pltpu.SemaphoreType.DMA,
            pltpu.SemaphoreType.DMA,
        ],
        compiler_params=params,
    )
    return presence, fill


def kernel(vessel_labels):
    presence_kernel, fill_kernel = _build_kernels()
    pres_part = presence_kernel(vessel_labels)
    present = jnp.any(pres_part[:, :_NIDS] > 0.0, axis=0)
    tabs = _tables(present)
    scal, oh = fill_kernel(vessel_labels, tabs)
    return scal, oh
